# baseline (speedup 1.0000x reference)
import functools

import jax
import jax.numpy as jnp
from jax import lax
from jax.experimental import pallas as pl
from jax.experimental.pallas import tpu as pltpu
from jax.experimental.pallas import tpu_sc as plsc

_VOCAB = 1000000
_EMBED = 64
_BATCH = 4096
_HIST = 50

_B_TOTAL = _BATCH * _HIST
_NC = 2
_NS = 16
_NW = _NC * _NS
_B_PER_W = _B_TOTAL // _NW
_CHUNK = 128
_N_CHUNKS = _B_PER_W // _CHUNK
_N_PAIRS = _N_CHUNKS // 2

_TBK = 4096
_TGRID = -(-_VOCAB // _TBK)

_mesh = plsc.VectorSubcoreMesh(core_axis_name="c", subcore_axis_name="s")


def _pack_body(t_ref, o_ref):
    o_ref[:, 0:_EMBED] = t_ref[...].T


_pack = pl.pallas_call(
    _pack_body,
    grid=(_TGRID,),
    in_specs=[pl.BlockSpec((_EMBED, _TBK), lambda i: (0, i))],
    out_specs=pl.BlockSpec((_TBK, 2 * _EMBED), lambda i: (i, 0)),
    out_shape=jax.ShapeDtypeStruct((_VOCAB, 2 * _EMBED), jnp.float32),
)


@functools.partial(
    pl.kernel,
    mesh=_mesh,
    out_type=jax.ShapeDtypeStruct((_B_TOTAL, _EMBED), jnp.float32),
    scratch_types=[
        pltpu.VMEM((_N_CHUNKS, _CHUNK), jnp.int32),
        pltpu.VMEM((_CHUNK, 2 * _EMBED), jnp.float32),
        pltpu.VMEM((_CHUNK, 2 * _EMBED), jnp.float32),
        pltpu.SemaphoreType.DMA,
        pltpu.SemaphoreType.DMA,
        pltpu.SemaphoreType.DMA,
        pltpu.SemaphoreType.DMA,
    ],
    compiler_params=pltpu.CompilerParams(use_tc_tiling_on_sc=False),
)
def _gather_kernel(idx_hbm, table_hbm, out_hbm, idx_v, buf_a, buf_b,
                   gsem_a, gsem_b, osem_a, osem_b):
    wid = lax.axis_index("s") * _NC + lax.axis_index("c")
    pltpu.sync_copy(idx_hbm.at[wid], idx_v)
    out_base = wid * _B_PER_W

    def _gather(c, buf, sem):
        return pltpu.async_copy(table_hbm.at[idx_v.at[c]], buf, sem)

    def _writeback(c, buf, sem):
        return pltpu.async_copy(
            buf.at[:, pl.ds(0, _EMBED)],
            out_hbm.at[pl.ds(out_base + c * _CHUNK, _CHUNK)], sem)

    def _drain_write(buf, sem):
        pltpu.make_async_copy(out_hbm.at[pl.ds(out_base, _CHUNK)],
                              buf.at[:, pl.ds(0, _EMBED)], sem).wait()

    def body(g, carry):
        ca = 2 * g
        cb = ca + 1

        ga = _gather(ca, buf_a, gsem_a)

        @pl.when(g > 0)
        def _():
            _drain_write(buf_b, osem_b)

        ga.wait()
        _writeback(ca, buf_a, osem_a)
        gb = _gather(cb, buf_b, gsem_b)

        @pl.when(g < _N_PAIRS - 1)
        def _():
            _drain_write(buf_a, osem_a)

        gb.wait()
        _writeback(cb, buf_b, osem_b)
        return carry

    lax.fori_loop(0, _N_PAIRS, body, 0)
    _drain_write(buf_a, osem_a)
    _drain_write(buf_b, osem_b)


def kernel(indices, table):
    idx = indices.T.astype(jnp.int32).reshape(_NW, _N_CHUNKS, _CHUNK)
    table_pack = _pack(table.T)
    out = _gather_kernel(idx, table_pack)
    return out.reshape(_HIST, _BATCH, _EMBED).transpose(1, 0, 2)


# speedup vs baseline: 1.0542x; 1.0542x over previous
import functools

import jax
import jax.numpy as jnp
from jax import lax
from jax.experimental import pallas as pl
from jax.experimental.pallas import tpu as pltpu
from jax.experimental.pallas import tpu_sc as plsc

_VOCAB = 1000000
_EMBED = 64
_BATCH = 4096
_HIST = 50

_B_TOTAL = _BATCH * _HIST
_NC = 2
_NS = 16
_NW = _NC * _NS
_B_PER_W = _B_TOTAL // _NW
_CHUNK = 128
_N_CHUNKS = _B_PER_W // _CHUNK
_N_PAIRS = _N_CHUNKS // 2

_HALF = 524288
_TBK = 4096
_TGRID = _HALF // _TBK
_LASTB = -(-_VOCAB // _TBK) - 1

_mesh = plsc.VectorSubcoreMesh(core_axis_name="c", subcore_axis_name="s")


def _pack_body(lo_ref, hi_ref, o_ref):
    o_ref[:, 0:_EMBED] = lo_ref[...].T
    o_ref[:, _EMBED:2 * _EMBED] = hi_ref[...].T


_pack = pl.pallas_call(
    _pack_body,
    grid=(_TGRID,),
    in_specs=[
        pl.BlockSpec((_EMBED, _TBK), lambda i: (0, i)),
        pl.BlockSpec((_EMBED, _TBK), lambda i: (0, jnp.minimum(i + _TGRID,
                                                               _LASTB))),
    ],
    out_specs=pl.BlockSpec((_TBK, 2 * _EMBED), lambda i: (i, 0)),
    out_shape=jax.ShapeDtypeStruct((_HALF, 2 * _EMBED), jnp.float32),
)


@functools.partial(
    pl.kernel,
    mesh=_mesh,
    out_type=jax.ShapeDtypeStruct((_B_TOTAL, 2 * _EMBED), jnp.float32),
    scratch_types=[
        pltpu.VMEM((_N_CHUNKS, _CHUNK), jnp.int32),
        pltpu.VMEM((_CHUNK, 2 * _EMBED), jnp.float32),
        pltpu.VMEM((_CHUNK, 2 * _EMBED), jnp.float32),
        pltpu.SemaphoreType.DMA,
        pltpu.SemaphoreType.DMA,
        pltpu.SemaphoreType.DMA,
        pltpu.SemaphoreType.DMA,
    ],
    compiler_params=pltpu.CompilerParams(use_tc_tiling_on_sc=True),
)
def _gather_kernel(idx_hbm, table_hbm, out_hbm, idx_v, buf_a, buf_b,
                   gsem_a, gsem_b, osem_a, osem_b):
    wid = lax.axis_index("s") * _NC + lax.axis_index("c")
    pltpu.sync_copy(idx_hbm.at[wid], idx_v)
    out_base = wid * _B_PER_W

    def _gather(c, buf, sem):
        return pltpu.async_copy(table_hbm.at[idx_v.at[c]], buf, sem)

    def _writeback(c, buf, sem):
        return pltpu.async_copy(
            buf, out_hbm.at[pl.ds(out_base + c * _CHUNK, _CHUNK)], sem)

    def _drain_write(buf, sem):
        pltpu.make_async_copy(out_hbm.at[pl.ds(out_base, _CHUNK)], buf,
                              sem).wait()

    def body(g, carry):
        ca = 2 * g
        cb = ca + 1

        ga = _gather(ca, buf_a, gsem_a)

        @pl.when(g > 0)
        def _():
            _drain_write(buf_b, osem_b)

        ga.wait()
        _writeback(ca, buf_a, osem_a)
        gb = _gather(cb, buf_b, gsem_b)

        @pl.when(g < _N_PAIRS - 1)
        def _():
            _drain_write(buf_a, osem_a)

        gb.wait()
        _writeback(cb, buf_b, osem_b)
        return carry

    lax.fori_loop(0, _N_PAIRS, body, 0)
    _drain_write(buf_a, osem_a)
    _drain_write(buf_b, osem_b)


def kernel(indices, table):
    flat = indices.T.astype(jnp.int32).reshape(_B_TOTAL)
    idx2 = (flat & (_HALF - 1)).reshape(_NW, _N_CHUNKS, _CHUNK)
    hi = (flat >= _HALF)[:, None]
    tT = table.T
    table_pack = _pack(tT, tT)
    out128 = _gather_kernel(idx2, table_pack)
    out = jnp.where(hi, out128[:, _EMBED:2 * _EMBED], out128[:, 0:_EMBED])
    return out.reshape(_HIST, _BATCH, _EMBED).transpose(1, 0, 2)


# speedup vs baseline: 1.1267x; 1.0688x over previous
import functools

import jax
import jax.numpy as jnp
from jax import lax
from jax.experimental import pallas as pl
from jax.experimental.pallas import tpu as pltpu
from jax.experimental.pallas import tpu_sc as plsc

_VOCAB = 1000000
_EMBED = 64
_BATCH = 4096
_HIST = 50

_B_TOTAL = _BATCH * _HIST
_NC = 2
_NS = 16
_NW = _NC * _NS
_B_PER_W = _B_TOTAL // _NW
_CHUNK = 128
_N_CHUNKS = _B_PER_W // _CHUNK
_N_PAIRS = _N_CHUNKS // 2

_HALF = 524288
_TBK = 8192
_TGRID = _HALF // _TBK
_LASTB = -(-_VOCAB // _TBK) - 1

_mesh = plsc.VectorSubcoreMesh(core_axis_name="c", subcore_axis_name="s")


def _pack_body(lo_ref, hi_ref, o_ref):
    o_ref[:, 0:_EMBED] = lo_ref[...].T
    o_ref[:, _EMBED:2 * _EMBED] = hi_ref[...].T


_pack = pl.pallas_call(
    _pack_body,
    grid=(_TGRID,),
    in_specs=[
        pl.BlockSpec((_EMBED, _TBK), lambda i: (0, i)),
        pl.BlockSpec((_EMBED, _TBK), lambda i: (0, jnp.minimum(i + _TGRID,
                                                               _LASTB))),
    ],
    out_specs=pl.BlockSpec((_TBK, 2 * _EMBED), lambda i: (i, 0)),
    out_shape=jax.ShapeDtypeStruct((_HALF, 2 * _EMBED), jnp.float32),
)


@functools.partial(
    pl.kernel,
    mesh=_mesh,
    out_type=jax.ShapeDtypeStruct((_B_TOTAL, 2 * _EMBED), jnp.float32),
    scratch_types=[
        pltpu.VMEM((_N_CHUNKS, _CHUNK), jnp.int32),
        pltpu.VMEM((_CHUNK, 2 * _EMBED), jnp.float32),
        pltpu.VMEM((_CHUNK, 2 * _EMBED), jnp.float32),
        pltpu.SemaphoreType.DMA,
        pltpu.SemaphoreType.DMA,
        pltpu.SemaphoreType.DMA,
        pltpu.SemaphoreType.DMA,
    ],
    compiler_params=pltpu.CompilerParams(use_tc_tiling_on_sc=True),
)
def _gather_kernel(idx_hbm, table_hbm, out_hbm, idx_v, buf_a, buf_b,
                   gsem_a, gsem_b, osem_a, osem_b):
    wid = lax.axis_index("s") * _NC + lax.axis_index("c")
    pltpu.sync_copy(idx_hbm.at[wid], idx_v)
    out_base = wid * _B_PER_W

    def _gather(c, buf, sem):
        return pltpu.async_copy(table_hbm.at[idx_v.at[c]], buf, sem)

    def _writeback(c, buf, sem):
        return pltpu.async_copy(
            buf, out_hbm.at[pl.ds(out_base + c * _CHUNK, _CHUNK)], sem)

    def _drain_write(buf, sem):
        pltpu.make_async_copy(out_hbm.at[pl.ds(out_base, _CHUNK)], buf,
                              sem).wait()

    def body(g, carry):
        ca = 2 * g
        cb = ca + 1

        ga = _gather(ca, buf_a, gsem_a)

        @pl.when(g > 0)
        def _():
            _drain_write(buf_b, osem_b)

        ga.wait()
        _writeback(ca, buf_a, osem_a)
        gb = _gather(cb, buf_b, gsem_b)

        @pl.when(g < _N_PAIRS - 1)
        def _():
            _drain_write(buf_a, osem_a)

        gb.wait()
        _writeback(cb, buf_b, osem_b)
        return carry

    lax.fori_loop(0, _N_PAIRS, body, 0)
    _drain_write(buf_a, osem_a)
    _drain_write(buf_b, osem_b)


def kernel(indices, table):
    flat = indices.T.astype(jnp.int32).reshape(_B_TOTAL)
    idx2 = (flat & (_HALF - 1)).reshape(_NW, _N_CHUNKS, _CHUNK)
    hi = (flat >= _HALF)[:, None]
    tT = table.T
    table_pack = _pack(tT, tT)
    out128 = _gather_kernel(idx2, table_pack)
    out = jnp.where(hi, out128[:, _EMBED:2 * _EMBED], out128[:, 0:_EMBED])
    return out.reshape(_HIST, _BATCH, _EMBED).transpose(1, 0, 2)


# speedup vs baseline: 1.1590x; 1.0287x over previous
import functools

import jax
import jax.numpy as jnp
from jax import lax
from jax.experimental import pallas as pl
from jax.experimental.pallas import tpu as pltpu
from jax.experimental.pallas import tpu_sc as plsc

_VOCAB = 1000000
_EMBED = 64
_BATCH = 4096
_HIST = 50

_B_TOTAL = _BATCH * _HIST
_NC = 2
_NS = 16
_NW = _NC * _NS
_B_PER_W = _B_TOTAL // _NW
_CHUNK = 128
_N_CHUNKS = _B_PER_W // _CHUNK
_N_PAIRS = _N_CHUNKS // 2

_HALF = 524288
_TBK = 16384
_TGRID = _HALF // _TBK
_LASTB = -(-_VOCAB // _TBK) - 1

_mesh = plsc.VectorSubcoreMesh(core_axis_name="c", subcore_axis_name="s")


def _pack_body(lo_ref, hi_ref, o_ref):
    o_ref[:, 0:_EMBED] = lo_ref[...].T
    o_ref[:, _EMBED:2 * _EMBED] = hi_ref[...].T


_pack = pl.pallas_call(
    _pack_body,
    grid=(_TGRID,),
    in_specs=[
        pl.BlockSpec((_EMBED, _TBK), lambda i: (0, i)),
        pl.BlockSpec((_EMBED, _TBK), lambda i: (0, jnp.minimum(i + _TGRID,
                                                               _LASTB))),
    ],
    out_specs=pl.BlockSpec((_TBK, 2 * _EMBED), lambda i: (i, 0)),
    out_shape=jax.ShapeDtypeStruct((_HALF, 2 * _EMBED), jnp.float32),
)


@functools.partial(
    pl.kernel,
    mesh=_mesh,
    out_type=jax.ShapeDtypeStruct((_B_TOTAL, 2 * _EMBED), jnp.float32),
    scratch_types=[
        pltpu.VMEM((_N_CHUNKS, _CHUNK), jnp.int32),
        pltpu.VMEM((_CHUNK, 2 * _EMBED), jnp.float32),
        pltpu.VMEM((_CHUNK, 2 * _EMBED), jnp.float32),
        pltpu.SemaphoreType.DMA,
        pltpu.SemaphoreType.DMA,
        pltpu.SemaphoreType.DMA,
        pltpu.SemaphoreType.DMA,
    ],
    compiler_params=pltpu.CompilerParams(use_tc_tiling_on_sc=True),
)
def _gather_kernel(idx_hbm, table_hbm, out_hbm, idx_v, buf_a, buf_b,
                   gsem_a, gsem_b, osem_a, osem_b):
    wid = lax.axis_index("s") * _NC + lax.axis_index("c")
    pltpu.sync_copy(idx_hbm.at[wid], idx_v)
    out_base = wid * _B_PER_W

    def _gather(c, buf, sem):
        return pltpu.async_copy(table_hbm.at[idx_v.at[c]], buf, sem)

    def _writeback(c, buf, sem):
        return pltpu.async_copy(
            buf, out_hbm.at[pl.ds(out_base + c * _CHUNK, _CHUNK)], sem)

    def _drain_write(buf, sem):
        pltpu.make_async_copy(out_hbm.at[pl.ds(out_base, _CHUNK)], buf,
                              sem).wait()

    def body(g, carry):
        ca = 2 * g
        cb = ca + 1

        ga = _gather(ca, buf_a, gsem_a)

        @pl.when(g > 0)
        def _():
            _drain_write(buf_b, osem_b)

        ga.wait()
        _writeback(ca, buf_a, osem_a)
        gb = _gather(cb, buf_b, gsem_b)

        @pl.when(g < _N_PAIRS - 1)
        def _():
            _drain_write(buf_a, osem_a)

        gb.wait()
        _writeback(cb, buf_b, osem_b)
        return carry

    lax.fori_loop(0, _N_PAIRS, body, 0)
    _drain_write(buf_a, osem_a)
    _drain_write(buf_b, osem_b)


def kernel(indices, table):
    flat = indices.T.astype(jnp.int32).reshape(_B_TOTAL)
    idx2 = (flat & (_HALF - 1)).reshape(_NW, _N_CHUNKS, _CHUNK)
    hi = (flat >= _HALF)[:, None]
    tT = table.T
    table_pack = _pack(tT, tT)
    out128 = _gather_kernel(idx2, table_pack)
    out = jnp.where(hi, out128[:, _EMBED:2 * _EMBED], out128[:, 0:_EMBED])
    return out.reshape(_HIST, _BATCH, _EMBED).transpose(1, 0, 2)
